# baseline (speedup 1.0000x reference)
import functools

import jax
import jax.numpy as jnp
from jax import lax
from jax.experimental import pallas as pl
from jax.experimental.pallas import tpu as pltpu
from jax.experimental.pallas import tpu_sc as plsc

N = 10000
E = 320000
HALF = E // 2
DT = 0.01
RB = 2000
EBP = 4000
EPS = 1e-5

NC = 2
NS = 16
NW = NC * NS
CH = 64
NQ = HALF // CH
QREM = NQ % NW
EPW = E // NW
PRW = HALF // NW
NFULL = PRW // CH
PTAIL = PRW - NFULL * CH
ZR = N // NS
GCH = 128
GNF = PRW // GCH
GTAIL = PRW - GNF * GCH


def _ln(y, g, b):
    m = jnp.mean(y, axis=-1, keepdims=True)
    v = jnp.mean((y - m) * (y - m), axis=-1, keepdims=True)
    return (y - m) * lax.rsqrt(v + EPS) * g + b


def _ln_p(y, g, b, gsum, gbc):
    m = _dot(_dot(y, gsum) * (1.0 / 64.0), gbc)
    d = y - m
    v = _dot(_dot(d * d, gsum) * (1.0 / 64.0), gbc)
    return d * lax.rsqrt(v + EPS) * g + b


def _dot(a, w):
    return jnp.dot(a, w, preferred_element_type=jnp.float32)


def _row_spec(cols):
    return pl.BlockSpec((RB, cols), lambda i: (i, 0))


def _erow_spec(cols):
    return pl.BlockSpec((EBP, cols), lambda i: (i, 0))


def _full_spec(shape):
    nd = len(shape)
    return pl.BlockSpec(shape, lambda i: (0,) * nd)


def _tc_call(body, grid, in_arrays, in_specs, out_shapes, out_specs):
    return pl.pallas_call(
        body,
        grid=(grid,),
        in_specs=in_specs,
        out_specs=out_specs,
        out_shape=out_shapes,
    )(*in_arrays)


def _encode_body(x_ref, w1, b1, w2, b2, g, b, w1s, w1d, eb1,
                 h_ref, a_ref, b_ref):
    x = x_ref[...]
    h = _ln(_dot(jax.nn.relu(_dot(x, w1[...]) + b1[...]), w2[...]) + b2[...],
            g[...], b[...])
    h_ref[...] = h
    a_ref[...] = _dot(h, w1s[...])
    b_ref[...] = _dot(h, w1d[...]) + eb1[...]


def _edge0_body(ealo_ref, eahi_ref, g_ref,
                ew1, eb1, ew2, eb2, eg, ebb,
                w1e, w2, b2, g, b, gsum, gbc,
                e_out):
    def enc(z):
        return _ln(_dot(jax.nn.relu(_dot(z, ew1[...]) + eb1[...]), ew2[...])
                   + eb2[...], eg[...], ebb[...])

    e0 = jnp.concatenate([enc(ealo_ref[...]), enc(eahi_ref[...])], axis=1)
    u = jax.nn.relu(g_ref[...] + _dot(e0, w1e[...]))
    v = _ln_p(_dot(u, w2[...]) + b2[...], g[...], b[...], gsum[...], gbc[...])
    e_out[...] = e0 + v


def _edge1_body(e_ref, g_ref,
                w1e, w2, b2, g, b, gsum, gbc,
                e_out):
    e0 = e_ref[...]
    u = jax.nn.relu(g_ref[...] + _dot(e0, w1e[...]))
    v = _ln_p(_dot(u, w2[...]) + b2[...], g[...], b[...], gsum[...], gbc[...])
    e_out[...] = e0 + v


def _node0_body(h_ref, p0_ref, p1_ref,
                w1a, w1b, b1, w2, b2, g, b,
                w1s, w1d, eb1,
                h_out, a_out, b_out):
    h = h_ref[...]
    agg = p0_ref[0] + p1_ref[0]
    u = jax.nn.relu(_dot(h, w1a[...]) + _dot(agg, w1b[...]) + b1[...])
    hn = h + _ln(_dot(u, w2[...]) + b2[...], g[...], b[...])
    h_out[...] = hn
    a_out[...] = _dot(hn, w1s[...])
    b_out[...] = _dot(hn, w1d[...]) + eb1[...]


def _final_body(h_ref, p0_ref, p1_ref, x_ref,
                w1a, w1b, b1, w2, b2, g, b,
                d1, db1, d2, db2,
                out_ref):
    h = h_ref[...]
    agg = p0_ref[0] + p1_ref[0]
    u = jax.nn.relu(_dot(h, w1a[...]) + _dot(agg, w1b[...]) + b1[...])
    h2 = h + _ln(_dot(u, w2[...]) + b2[...], g[...], b[...])
    o = _dot(jax.nn.relu(_dot(h2, d1[...]) + db1[...]), d2[...]) + db2[...]
    x = x_ref[...]
    mask = (x[:, 1:2] <= x[:, 2:3] + DT).astype(jnp.float32)
    out_ref[...] = o * mask


def _sc_mesh():
    return plsc.VectorSubcoreMesh(core_axis_name="c", subcore_axis_name="s")


def _gather_ab(a_tab, b_tab, src, dst):
    @functools.partial(
        pl.kernel,
        out_type=jax.ShapeDtypeStruct((HALF, 128), jnp.float32),
        mesh=_sc_mesh(),
        compiler_params=pltpu.CompilerParams(use_tc_tiling_on_sc=False),
        scratch_types=[
            pltpu.VMEM((PRW,), jnp.int32),
            pltpu.VMEM((PRW,), jnp.int32),
            pltpu.VMEM((PRW,), jnp.int32),
            pltpu.VMEM((PRW,), jnp.int32),
            pltpu.VMEM((GCH, 64), jnp.float32),
            pltpu.VMEM((GCH, 64), jnp.float32),
            pltpu.VMEM((GCH, 64), jnp.float32),
            pltpu.VMEM((GCH, 64), jnp.float32),
            pltpu.VMEM((GCH, 128), jnp.float32),
            pltpu.SemaphoreType.DMA,
            pltpu.SemaphoreType.DMA,
            pltpu.SemaphoreType.DMA,
            pltpu.SemaphoreType.DMA,
        ],
    )
    def gk(a_hbm, b_hbm, src_hbm, dst_hbm, g_hbm,
           qls, qld, qhs, qhd, ta, tb, tc, td, rc, s1, s2, s3, s4):
        cid = lax.axis_index("c")
        sid = lax.axis_index("s")
        wid = cid * NS + sid
        base = wid * PRW
        pltpu.sync_copy(src_hbm.at[pl.ds(base, PRW)], qls)
        pltpu.sync_copy(dst_hbm.at[pl.ds(base, PRW)], qld)
        pltpu.sync_copy(src_hbm.at[pl.ds(HALF + base, PRW)], qhs)
        pltpu.sync_copy(dst_hbm.at[pl.ds(HALF + base, PRW)], qhd)

        def chunk(off, n):
            c1 = pltpu.async_copy(a_hbm.at[qls.at[pl.ds(off, n)]],
                                  ta.at[pl.ds(0, n)], s1)
            c2 = pltpu.async_copy(b_hbm.at[qld.at[pl.ds(off, n)]],
                                  tb.at[pl.ds(0, n)], s2)
            c3 = pltpu.async_copy(a_hbm.at[qhs.at[pl.ds(off, n)]],
                                  tc.at[pl.ds(0, n)], s3)
            c4 = pltpu.async_copy(b_hbm.at[qhd.at[pl.ds(off, n)]],
                                  td.at[pl.ds(0, n)], s4)
            c1.wait()
            c2.wait()
            c3.wait()
            c4.wait()

            def row(r, c2_):
                for cc in range(4):
                    sl = pl.ds(cc * 16, 16)
                    rc[r, sl] = ta[r, sl] + tb[r, sl]
                    rc[r, pl.ds(64 + cc * 16, 16)] = tc[r, sl] + td[r, sl]
                return c2_

            lax.fori_loop(0, n, row, 0)
            pltpu.sync_copy(rc.at[pl.ds(0, n)], g_hbm.at[pl.ds(base + off, n)])

        def body(j, carry):
            chunk(j * GCH, GCH)
            return carry

        lax.fori_loop(0, GNF, body, 0)
        if GTAIL:
            chunk(GNF * GCH, GTAIL)

    return gk(a_tab, b_tab, src, dst)


def _segment_partials(e, dst):
    @functools.partial(
        pl.kernel,
        out_type=jax.ShapeDtypeStruct((NC, N, 64), jnp.float32),
        mesh=_sc_mesh(),
        compiler_params=pltpu.CompilerParams(use_tc_tiling_on_sc=False),
        scratch_types=[
            pltpu.VMEM((2 * CH,), jnp.int32),
            pltpu.VMEM((2 * PTAIL,), jnp.int32),
            pltpu.VMEM((2 * CH, 64), jnp.float32),
            pltpu.VMEM((2 * PTAIL, 64), jnp.float32),
            pltpu.VMEM((CH, 128), jnp.float32),
            pltpu.VMEM((ZR, 64), jnp.float32),
            pltpu.VMEM_SHARED((N, 64), jnp.float32),
        ],
    )
    def sk(e_hbm, dst_hbm, out_hbm, idxc, idxt, ebuf, ebuft, epk, zbuf, agg):
        cid = lax.axis_index("c")
        sid = lax.axis_index("s")
        wid = cid * NS + sid
        base = wid * PRW

        def zrow(r, carry):
            for cc in range(4):
                zbuf[r, pl.ds(cc * 16, 16)] = jnp.zeros((16,), jnp.float32)
            return carry

        lax.fori_loop(0, ZR, zrow, 0)
        pltpu.sync_copy(zbuf, agg.at[pl.ds(sid * ZR, ZR)])
        plsc.subcore_barrier()

        def body(j, carry):
            off = base + j * CH
            pltpu.sync_copy(dst_hbm.at[pl.ds(off, CH)], idxc.at[pl.ds(0, CH)])
            pltpu.sync_copy(dst_hbm.at[pl.ds(HALF + off, CH)],
                            idxc.at[pl.ds(CH, CH)])
            pltpu.sync_copy(e_hbm.at[pl.ds(off, CH)], epk)

            def row(r, c2):
                for cc in range(4):
                    sl = pl.ds(cc * 16, 16)
                    ebuf[r, sl] = epk[r, sl]
                    ebuf[CH + r, sl] = epk[r, pl.ds(64 + cc * 16, 16)]
                return c2

            lax.fori_loop(0, CH, row, 0)
            pltpu.sync_copy(ebuf, agg.at[idxc], add=True)
            return carry

        lax.fori_loop(0, NFULL, body, 0)
        if PTAIL:
            off = base + NFULL * CH
            pltpu.sync_copy(dst_hbm.at[pl.ds(off, PTAIL)], idxt.at[pl.ds(0, PTAIL)])
            pltpu.sync_copy(dst_hbm.at[pl.ds(HALF + off, PTAIL)],
                            idxt.at[pl.ds(PTAIL, PTAIL)])
            pltpu.sync_copy(e_hbm.at[pl.ds(off, PTAIL)], epk.at[pl.ds(0, PTAIL)])

            def trow(r, c2):
                for cc in range(4):
                    sl = pl.ds(cc * 16, 16)
                    ebuft[r, sl] = epk[r, sl]
                    ebuft[PTAIL + r, sl] = epk[r, pl.ds(64 + cc * 16, 16)]
                return c2

            lax.fori_loop(0, PTAIL, trow, 0)
            pltpu.sync_copy(ebuft, agg.at[idxt], add=True)

        plsc.subcore_barrier()

        @pl.when(sid == 0)
        def _():
            pltpu.sync_copy(agg, out_hbm.at[cid])

    return sk(e, dst)


def kernel(x, edge_index, edge_attr, params):
    src = edge_index[0]
    dst = edge_index[1]
    enc_n, enc_e, dec = params["enc_n"], params["enc_e"], params["dec"]
    l0, l1 = params["layers"][0], params["layers"][1]

    def r2(a):
        return a.reshape(1, -1)

    f32 = jnp.float32
    sd64 = jax.ShapeDtypeStruct((N, 64), f32)
    sd128 = jax.ShapeDtypeStruct((N, 128), f32)
    sep = jax.ShapeDtypeStruct((HALF, 128), f32)

    ew1s = {}
    for i, lp in enumerate((l0, l1)):
        w = lp["edge"]["W1"]
        ew1s[i] = (w[:64], w[64:128], w[128:])
    nw1s = {i: (lp["node"]["W1"][:64], lp["node"]["W1"][64:])
            for i, lp in enumerate((l0, l1))}

    def bd(w):
        z = jnp.zeros_like(w)
        return jnp.concatenate(
            [jnp.concatenate([w, z], axis=1), jnp.concatenate([z, w], axis=1)],
            axis=0)

    def p2(v):
        return jnp.concatenate([v, v]).reshape(1, 128)

    gsum = jnp.concatenate(
        [jnp.concatenate([jnp.ones((64, 1), f32), jnp.zeros((64, 1), f32)], axis=1),
         jnp.concatenate([jnp.zeros((64, 1), f32), jnp.ones((64, 1), f32)], axis=1)],
        axis=0)
    gbc = gsum.T

    h0, a0, b0 = _tc_call(
        _encode_body, N // RB,
        [x, enc_n["W1"], r2(enc_n["b1"]), enc_n["W2"], r2(enc_n["b2"]),
         r2(enc_n["g"]), r2(enc_n["b"]),
         ew1s[0][0], ew1s[0][1], r2(l0["edge"]["b1"])],
        [_row_spec(128)] + [_full_spec(s.shape) for s in
                            (enc_n["W1"], r2(enc_n["b1"]), enc_n["W2"],
                             r2(enc_n["b2"]), r2(enc_n["g"]), r2(enc_n["b"]),
                             ew1s[0][0], ew1s[0][1], r2(l0["edge"]["b1"]))],
        (sd64, sd64, sd64),
        (_row_spec(64), _row_spec(64), _row_spec(64)),
    )

    g0 = _gather_ab(a0, b0, src, dst)

    ew0 = (enc_e["W1"], r2(enc_e["b1"]), enc_e["W2"], r2(enc_e["b2"]),
           r2(enc_e["g"]), r2(enc_e["b"]),
           bd(ew1s[0][2]), bd(l0["edge"]["W2"]), p2(l0["edge"]["b2"]),
           p2(l0["edge"]["g"]), p2(l0["edge"]["b"]), gsum, gbc)
    nbe = HALF // EBP
    e1 = _tc_call(
        _edge0_body, nbe,
        [edge_attr, edge_attr, g0] + list(ew0),
        [pl.BlockSpec((EBP, 2), lambda i: (i, 0)),
         pl.BlockSpec((EBP, 2), lambda i: (i + nbe, 0)),
         _erow_spec(128)] +
        [_full_spec(s.shape) for s in ew0],
        sep, _erow_spec(128),
    )

    parts0 = _segment_partials(e1, dst)

    pspec0 = pl.BlockSpec((1, RB, 64), lambda i: (0, i, 0))
    pspec1 = pl.BlockSpec((1, RB, 64), lambda i: (1, i, 0))

    h1, a1, b1t = _tc_call(
        _node0_body, N // RB,
        [h0, parts0, parts0,
         nw1s[0][0], nw1s[0][1], r2(l0["node"]["b1"]),
         l0["node"]["W2"], r2(l0["node"]["b2"]),
         r2(l0["node"]["g"]), r2(l0["node"]["b"]),
         ew1s[1][0], ew1s[1][1], r2(l1["edge"]["b1"])],
        [_row_spec(64), pspec0, pspec1] +
        [_full_spec(s.shape) for s in
         (nw1s[0][0], nw1s[0][1], r2(l0["node"]["b1"]),
          l0["node"]["W2"], r2(l0["node"]["b2"]),
          r2(l0["node"]["g"]), r2(l0["node"]["b"]),
          ew1s[1][0], ew1s[1][1], r2(l1["edge"]["b1"]))],
        (sd64, sd64, sd64),
        (_row_spec(64), _row_spec(64), _row_spec(64)),
    )

    g1 = _gather_ab(a1, b1t, src, dst)

    ew1 = (bd(ew1s[1][2]), bd(l1["edge"]["W2"]), p2(l1["edge"]["b2"]),
           p2(l1["edge"]["g"]), p2(l1["edge"]["b"]), gsum, gbc)
    e2 = _tc_call(
        _edge1_body, nbe,
        [e1, g1] + list(ew1),
        [_erow_spec(128)] * 2 +
        [_full_spec(s.shape) for s in ew1],
        sep, _erow_spec(128),
    )

    parts1 = _segment_partials(e2, dst)

    out = _tc_call(
        _final_body, N // RB,
        [h1, parts1, parts1, x,
         nw1s[1][0], nw1s[1][1], r2(l1["node"]["b1"]),
         l1["node"]["W2"], r2(l1["node"]["b2"]),
         r2(l1["node"]["g"]), r2(l1["node"]["b"]),
         dec["W1"], r2(dec["b1"]), dec["W2"], r2(dec["b2"])],
        [_row_spec(64), pspec0, pspec1, _row_spec(128)] +
        [_full_spec(s.shape) for s in
         (nw1s[1][0], nw1s[1][1], r2(l1["node"]["b1"]),
          l1["node"]["W2"], r2(l1["node"]["b2"]),
          r2(l1["node"]["g"]), r2(l1["node"]["b"]),
          dec["W1"], r2(dec["b1"]), dec["W2"], r2(dec["b2"]))],
        jax.ShapeDtypeStruct((N, 128), f32),
        _row_spec(128),
    )
    return out


# speedup vs baseline: 1.5482x; 1.5482x over previous
import functools

import jax
import jax.numpy as jnp
from jax import lax
from jax.experimental import pallas as pl
from jax.experimental.pallas import tpu as pltpu
from jax.experimental.pallas import tpu_sc as plsc

N = 10000
E = 320000
HALF = E // 2
DT = 0.01
RB = 2000
EBP = 4000
EPS = 1e-5

NC = 2
NS = 16
NW = NC * NS
CH = 64
NQ = HALF // CH
QREM = NQ % NW
EPW = E // NW
PRW = HALF // NW
NFULL = PRW // CH
PTAIL = PRW - NFULL * CH
ZR = N // NS
GCH = 64
GNF = PRW // GCH
GTAIL = PRW - GNF * GCH


def _ln(y, g, b):
    m = jnp.mean(y, axis=-1, keepdims=True)
    v = jnp.mean((y - m) * (y - m), axis=-1, keepdims=True)
    return (y - m) * lax.rsqrt(v + EPS) * g + b


def _ln_p(y, g, b, gsum, gbc):
    m = _dot(_dot(y, gsum) * (1.0 / 64.0), gbc)
    d = y - m
    v = _dot(_dot(d * d, gsum) * (1.0 / 64.0), gbc)
    return d * lax.rsqrt(v + EPS) * g + b


def _dot(a, w):
    return jnp.dot(a, w, preferred_element_type=jnp.float32)


def _row_spec(cols):
    return pl.BlockSpec((RB, cols), lambda i: (i, 0))


def _erow_spec(cols):
    return pl.BlockSpec((EBP, cols), lambda i: (i, 0))


def _full_spec(shape):
    nd = len(shape)
    return pl.BlockSpec(shape, lambda i: (0,) * nd)


def _tc_call(body, grid, in_arrays, in_specs, out_shapes, out_specs):
    return pl.pallas_call(
        body,
        grid=(grid,),
        in_specs=in_specs,
        out_specs=out_specs,
        out_shape=out_shapes,
    )(*in_arrays)


def _encode_body(x_ref, w1, b1, w2, b2, g, b, w1s, w1d, eb1,
                 h_ref, a_ref, b_ref):
    x = x_ref[...]
    h = _ln(_dot(jax.nn.relu(_dot(x, w1[...]) + b1[...]), w2[...]) + b2[...],
            g[...], b[...])
    h_ref[...] = h
    a_ref[...] = _dot(h, w1s[...])
    b_ref[...] = _dot(h, w1d[...]) + eb1[...]


def _edge0_body(ealo_ref, eahi_ref, g_ref,
                ew1, eb1, ew2, eb2, eg, ebb,
                w1e, w2, b2, g, b, gsum, gbc,
                e_out):
    ea = jnp.concatenate([ealo_ref[...], eahi_ref[...]], axis=1)
    e0 = _ln_p(_dot(jax.nn.relu(_dot(ea, ew1[...]) + eb1[...]), ew2[...])
               + eb2[...], eg[...], ebb[...], gsum[...], gbc[...])
    u = jax.nn.relu(g_ref[...] + _dot(e0, w1e[...]))
    v = _ln_p(_dot(u, w2[...]) + b2[...], g[...], b[...], gsum[...], gbc[...])
    e_out[...] = e0 + v


def _edge1_body(e_ref, g_ref,
                w1e, w2, b2, g, b, gsum, gbc,
                e_out):
    e0 = e_ref[...]
    u = jax.nn.relu(g_ref[...] + _dot(e0, w1e[...]))
    v = _ln_p(_dot(u, w2[...]) + b2[...], g[...], b[...], gsum[...], gbc[...])
    e_out[...] = e0 + v


def _node0_body(h_ref, p0_ref, p1_ref,
                w1a, w1b, b1, w2, b2, g, b,
                w1s, w1d, eb1,
                h_out, a_out, b_out):
    h = h_ref[...]
    agg = p0_ref[0] + p1_ref[0]
    u = jax.nn.relu(_dot(h, w1a[...]) + _dot(agg, w1b[...]) + b1[...])
    hn = h + _ln(_dot(u, w2[...]) + b2[...], g[...], b[...])
    h_out[...] = hn
    a_out[...] = _dot(hn, w1s[...])
    b_out[...] = _dot(hn, w1d[...]) + eb1[...]


def _final_body(h_ref, p0_ref, p1_ref, x_ref,
                w1a, w1b, b1, w2, b2, g, b,
                d1, db1, d2, db2,
                out_ref):
    h = h_ref[...]
    agg = p0_ref[0] + p1_ref[0]
    u = jax.nn.relu(_dot(h, w1a[...]) + _dot(agg, w1b[...]) + b1[...])
    h2 = h + _ln(_dot(u, w2[...]) + b2[...], g[...], b[...])
    o = _dot(jax.nn.relu(_dot(h2, d1[...]) + db1[...]), d2[...]) + db2[...]
    x = x_ref[...]
    mask = (x[:, 1:2] <= x[:, 2:3] + DT).astype(jnp.float32)
    out_ref[...] = o * mask


def _sc_mesh():
    return plsc.VectorSubcoreMesh(core_axis_name="c", subcore_axis_name="s")


def _gather_ab(a_tab, b_tab, src, dst):
    @functools.partial(
        pl.kernel,
        out_type=jax.ShapeDtypeStruct((HALF, 128), jnp.float32),
        mesh=_sc_mesh(),
        compiler_params=pltpu.CompilerParams(use_tc_tiling_on_sc=False),
        scratch_types=[
            pltpu.VMEM((PRW,), jnp.int32),
            pltpu.VMEM((PRW,), jnp.int32),
            pltpu.VMEM((PRW,), jnp.int32),
            pltpu.VMEM((PRW,), jnp.int32),
            [pltpu.VMEM((GCH, 64), jnp.float32)] * 4,
            [pltpu.VMEM((GCH, 64), jnp.float32)] * 4,
            pltpu.VMEM((GCH, 128), jnp.float32),
            pltpu.VMEM((GCH, 128), jnp.float32),
            pltpu.SemaphoreType.DMA,
            pltpu.SemaphoreType.DMA,
        ],
    )
    def gk(a_hbm, b_hbm, src_hbm, dst_hbm, g_hbm,
           qls, qld, qhs, qhd, bufs0, bufs1, rc0, rc1, sem0, sem1):
        cid = lax.axis_index("c")
        sid = lax.axis_index("s")
        wid = cid * NS + sid
        base = wid * PRW
        pltpu.sync_copy(src_hbm.at[pl.ds(base, PRW)], qls)
        pltpu.sync_copy(dst_hbm.at[pl.ds(base, PRW)], qld)
        pltpu.sync_copy(src_hbm.at[pl.ds(HALF + base, PRW)], qhs)
        pltpu.sync_copy(dst_hbm.at[pl.ds(HALF + base, PRW)], qhd)

        idxs = (qls, qld, qhs, qhd)
        tabs = (a_hbm, b_hbm, a_hbm, b_hbm)
        sems = (sem0, sem1)
        bufsets = (bufs0, bufs1)
        rcs = (rc0, rc1)

        def start(off, bs):
            for k in range(4):
                pltpu.async_copy(tabs[k].at[idxs[k].at[pl.ds(off, GCH)]],
                                 bufsets[bs][k], sems[bs])

        def finish(off, bs):
            for k in range(4):
                pltpu.make_async_copy(tabs[k].at[idxs[k].at[pl.ds(off, GCH)]],
                                      bufsets[bs][k], sems[bs]).wait()
            ta, tb, tc, td = bufsets[bs]
            rc = rcs[bs]

            def row(r, c2_):
                for cc in range(4):
                    sl = pl.ds(cc * 16, 16)
                    rc[r, sl] = ta[r, sl] + tb[r, sl]
                    rc[r, pl.ds(64 + cc * 16, 16)] = tc[r, sl] + td[r, sl]
                return c2_

            lax.fori_loop(0, GCH, row, 0)
            pltpu.sync_copy(rc, g_hbm.at[pl.ds(base + off, GCH)])

        start(0, 0)

        def outer(j, carry):
            start((2 * j + 1) * GCH, 1)
            finish(2 * j * GCH, 0)

            @pl.when(j < GNF // 2 - 1)
            def _():
                start((2 * j + 2) * GCH, 0)

            finish((2 * j + 1) * GCH, 1)
            return carry

        lax.fori_loop(0, GNF // 2, outer, 0)

        if GTAIL:
            off = GNF * GCH
            for k in range(4):
                pltpu.async_copy(
                    tabs[k].at[idxs[k].at[pl.ds(off, GTAIL)]],
                    bufsets[0][k].at[pl.ds(0, GTAIL)], sems[0])
            for k in range(4):
                pltpu.make_async_copy(
                    tabs[k].at[idxs[k].at[pl.ds(off, GTAIL)]],
                    bufsets[0][k].at[pl.ds(0, GTAIL)], sems[0]).wait()
            ta, tb, tc, td = bufsets[0]
            rc = rcs[0]

            def trow(r, c2_):
                for cc in range(4):
                    sl = pl.ds(cc * 16, 16)
                    rc[r, sl] = ta[r, sl] + tb[r, sl]
                    rc[r, pl.ds(64 + cc * 16, 16)] = tc[r, sl] + td[r, sl]
                return c2_

            lax.fori_loop(0, GTAIL, trow, 0)
            pltpu.sync_copy(rc.at[pl.ds(0, GTAIL)],
                            g_hbm.at[pl.ds(base + off, GTAIL)])

    return gk(a_tab, b_tab, src, dst)


def _segment_partials(e, dst):
    @functools.partial(
        pl.kernel,
        out_type=jax.ShapeDtypeStruct((NC, N, 64), jnp.float32),
        mesh=_sc_mesh(),
        compiler_params=pltpu.CompilerParams(use_tc_tiling_on_sc=False),
        scratch_types=[
            pltpu.VMEM((2 * CH,), jnp.int32),
            pltpu.VMEM((2 * CH,), jnp.int32),
            pltpu.VMEM((CH, 128), jnp.float32),
            pltpu.VMEM((CH, 128), jnp.float32),
            pltpu.VMEM((2 * CH, 64), jnp.float32),
            pltpu.VMEM((2 * CH, 64), jnp.float32),
            pltpu.VMEM((2 * PTAIL,), jnp.int32),
            pltpu.VMEM((ZR, 64), jnp.float32),
            pltpu.VMEM_SHARED((N, 64), jnp.float32),
            pltpu.SemaphoreType.DMA,
            pltpu.SemaphoreType.DMA,
        ],
    )
    def sk(e_hbm, dst_hbm, out_hbm, idx0, idx1, epk0, epk1, eb0, eb1,
           idxt, zbuf, agg, sem0, sem1):
        cid = lax.axis_index("c")
        sid = lax.axis_index("s")
        wid = cid * NS + sid
        base = wid * PRW

        def zrow(r, carry):
            for cc in range(4):
                zbuf[r, pl.ds(cc * 16, 16)] = jnp.zeros((16,), jnp.float32)
            return carry

        lax.fori_loop(0, ZR, zrow, 0)
        pltpu.sync_copy(zbuf, agg.at[pl.ds(sid * ZR, ZR)])
        plsc.subcore_barrier()

        idxs = (idx0, idx1)
        epks = (epk0, epk1)
        ebufs = (eb0, eb1)
        sems = (sem0, sem1)

        def start(off, bs):
            pltpu.async_copy(dst_hbm.at[pl.ds(off, CH)],
                             idxs[bs].at[pl.ds(0, CH)], sems[bs])
            pltpu.async_copy(dst_hbm.at[pl.ds(HALF + off, CH)],
                             idxs[bs].at[pl.ds(CH, CH)], sems[bs])
            pltpu.async_copy(e_hbm.at[pl.ds(off, CH)], epks[bs], sems[bs])

        def finish(off, bs):
            pltpu.make_async_copy(dst_hbm.at[pl.ds(off, CH)],
                                  idxs[bs].at[pl.ds(0, CH)], sems[bs]).wait()
            pltpu.make_async_copy(dst_hbm.at[pl.ds(HALF + off, CH)],
                                  idxs[bs].at[pl.ds(CH, CH)], sems[bs]).wait()
            pltpu.make_async_copy(e_hbm.at[pl.ds(off, CH)], epks[bs],
                                  sems[bs]).wait()
            epk = epks[bs]
            ebuf = ebufs[bs]

            def row(r, c2):
                for cc in range(4):
                    sl = pl.ds(cc * 16, 16)
                    ebuf[r, sl] = epk[r, sl]
                    ebuf[CH + r, sl] = epk[r, pl.ds(64 + cc * 16, 16)]
                return c2

            lax.fori_loop(0, CH, row, 0)
            pltpu.sync_copy(ebuf, agg.at[idxs[bs]], add=True)

        start(base, 0)

        def outer(j, carry):
            start(base + (2 * j + 1) * CH, 1)
            finish(base + 2 * j * CH, 0)

            @pl.when(j < NFULL // 2 - 1)
            def _():
                start(base + (2 * j + 2) * CH, 0)

            finish(base + (2 * j + 1) * CH, 1)
            return carry

        lax.fori_loop(0, NFULL // 2, outer, 0)

        if PTAIL:
            off = base + NFULL * CH
            pltpu.sync_copy(dst_hbm.at[pl.ds(off, PTAIL)], idxt.at[pl.ds(0, PTAIL)])
            pltpu.sync_copy(dst_hbm.at[pl.ds(HALF + off, PTAIL)],
                            idxt.at[pl.ds(PTAIL, PTAIL)])
            pltpu.sync_copy(e_hbm.at[pl.ds(off, PTAIL)], epk0.at[pl.ds(0, PTAIL)])

            def trow(r, c2):
                for cc in range(4):
                    sl = pl.ds(cc * 16, 16)
                    eb0[r, sl] = epk0[r, sl]
                    eb0[PTAIL + r, sl] = epk0[r, pl.ds(64 + cc * 16, 16)]
                return c2

            lax.fori_loop(0, PTAIL, trow, 0)
            pltpu.sync_copy(eb0.at[pl.ds(0, 2 * PTAIL)], agg.at[idxt], add=True)

        plsc.subcore_barrier()

        @pl.when(sid == 0)
        def _():
            pltpu.sync_copy(agg, out_hbm.at[cid])

    return sk(e, dst)


def kernel(x, edge_index, edge_attr, params):
    src = edge_index[0]
    dst = edge_index[1]
    enc_n, enc_e, dec = params["enc_n"], params["enc_e"], params["dec"]
    l0, l1 = params["layers"][0], params["layers"][1]

    def r2(a):
        return a.reshape(1, -1)

    f32 = jnp.float32
    sd64 = jax.ShapeDtypeStruct((N, 64), f32)
    sd128 = jax.ShapeDtypeStruct((N, 128), f32)
    sep = jax.ShapeDtypeStruct((HALF, 128), f32)

    ew1s = {}
    for i, lp in enumerate((l0, l1)):
        w = lp["edge"]["W1"]
        ew1s[i] = (w[:64], w[64:128], w[128:])
    nw1s = {i: (lp["node"]["W1"][:64], lp["node"]["W1"][64:])
            for i, lp in enumerate((l0, l1))}

    def bd(w):
        z = jnp.zeros_like(w)
        return jnp.concatenate(
            [jnp.concatenate([w, z], axis=1), jnp.concatenate([z, w], axis=1)],
            axis=0)

    def p2(v):
        return jnp.concatenate([v, v]).reshape(1, 128)

    gsum = jnp.concatenate(
        [jnp.concatenate([jnp.ones((64, 1), f32), jnp.zeros((64, 1), f32)], axis=1),
         jnp.concatenate([jnp.zeros((64, 1), f32), jnp.ones((64, 1), f32)], axis=1)],
        axis=0)
    gbc = gsum.T

    h0, a0, b0 = _tc_call(
        _encode_body, N // RB,
        [x, enc_n["W1"], r2(enc_n["b1"]), enc_n["W2"], r2(enc_n["b2"]),
         r2(enc_n["g"]), r2(enc_n["b"]),
         ew1s[0][0], ew1s[0][1], r2(l0["edge"]["b1"])],
        [_row_spec(128)] + [_full_spec(s.shape) for s in
                            (enc_n["W1"], r2(enc_n["b1"]), enc_n["W2"],
                             r2(enc_n["b2"]), r2(enc_n["g"]), r2(enc_n["b"]),
                             ew1s[0][0], ew1s[0][1], r2(l0["edge"]["b1"]))],
        (sd64, sd64, sd64),
        (_row_spec(64), _row_spec(64), _row_spec(64)),
    )

    g0 = _gather_ab(a0, b0, src, dst)

    ew0 = (bd(enc_e["W1"]), p2(enc_e["b1"]), bd(enc_e["W2"]), p2(enc_e["b2"]),
           p2(enc_e["g"]), p2(enc_e["b"]),
           bd(ew1s[0][2]), bd(l0["edge"]["W2"]), p2(l0["edge"]["b2"]),
           p2(l0["edge"]["g"]), p2(l0["edge"]["b"]), gsum, gbc)
    nbe = HALF // EBP
    e1 = _tc_call(
        _edge0_body, nbe,
        [edge_attr, edge_attr, g0] + list(ew0),
        [pl.BlockSpec((EBP, 2), lambda i: (i, 0)),
         pl.BlockSpec((EBP, 2), lambda i: (i + nbe, 0)),
         _erow_spec(128)] +
        [_full_spec(s.shape) for s in ew0],
        sep, _erow_spec(128),
    )

    parts0 = _segment_partials(e1, dst)

    pspec0 = pl.BlockSpec((1, RB, 64), lambda i: (0, i, 0))
    pspec1 = pl.BlockSpec((1, RB, 64), lambda i: (1, i, 0))

    h1, a1, b1t = _tc_call(
        _node0_body, N // RB,
        [h0, parts0, parts0,
         nw1s[0][0], nw1s[0][1], r2(l0["node"]["b1"]),
         l0["node"]["W2"], r2(l0["node"]["b2"]),
         r2(l0["node"]["g"]), r2(l0["node"]["b"]),
         ew1s[1][0], ew1s[1][1], r2(l1["edge"]["b1"])],
        [_row_spec(64), pspec0, pspec1] +
        [_full_spec(s.shape) for s in
         (nw1s[0][0], nw1s[0][1], r2(l0["node"]["b1"]),
          l0["node"]["W2"], r2(l0["node"]["b2"]),
          r2(l0["node"]["g"]), r2(l0["node"]["b"]),
          ew1s[1][0], ew1s[1][1], r2(l1["edge"]["b1"]))],
        (sd64, sd64, sd64),
        (_row_spec(64), _row_spec(64), _row_spec(64)),
    )

    g1 = _gather_ab(a1, b1t, src, dst)

    ew1 = (bd(ew1s[1][2]), bd(l1["edge"]["W2"]), p2(l1["edge"]["b2"]),
           p2(l1["edge"]["g"]), p2(l1["edge"]["b"]), gsum, gbc)
    e2 = _tc_call(
        _edge1_body, nbe,
        [e1, g1] + list(ew1),
        [_erow_spec(128)] * 2 +
        [_full_spec(s.shape) for s in ew1],
        sep, _erow_spec(128),
    )

    parts1 = _segment_partials(e2, dst)

    out = _tc_call(
        _final_body, N // RB,
        [h1, parts1, parts1, x,
         nw1s[1][0], nw1s[1][1], r2(l1["node"]["b1"]),
         l1["node"]["W2"], r2(l1["node"]["b2"]),
         r2(l1["node"]["g"]), r2(l1["node"]["b"]),
         dec["W1"], r2(dec["b1"]), dec["W2"], r2(dec["b2"])],
        [_row_spec(64), pspec0, pspec1, _row_spec(128)] +
        [_full_spec(s.shape) for s in
         (nw1s[1][0], nw1s[1][1], r2(l1["node"]["b1"]),
          l1["node"]["W2"], r2(l1["node"]["b2"]),
          r2(l1["node"]["g"]), r2(l1["node"]["b"]),
          dec["W1"], r2(dec["b1"]), dec["W2"], r2(dec["b2"]))],
        jax.ShapeDtypeStruct((N, 128), f32),
        _row_spec(128),
    )
    return out
